# Initial kernel scaffold; baseline (speedup 1.0000x reference)
import jax
import jax.numpy as jnp
from jax.experimental import pallas as pl


def kernel(inputs, shared_weights):
    raise NotImplementedError("write your pallas kernel here")


# speedup vs baseline: 1.1098x; 1.1098x over previous
import functools

import jax
import jax.numpy as jnp
from jax import lax
from jax.experimental import pallas as pl
from jax.experimental.pallas import tpu as pltpu
from jax.experimental.pallas import tpu_sc as plsc

VOCAB_SIZE = 1000000
EMBEDDING_DIM = 32
BATCH = 16384
SEQ_LEN = 50

_NC = 2
_NS = 16
_NW = _NC * _NS

_TOTAL = BATCH * SEQ_LEN
_CHUNK = 128
_CPT = _TOTAL // (_NW * _CHUNK)
_GROUP = 8
_NGROUP = _CPT // _GROUP


def _body(table_hbm, idx_hbm, out_hbm, idx_v, rows_v, gsems, csems):
    wid = lax.axis_index("s") * _NC + lax.axis_index("c")
    crow0 = wid * _CPT

    pltpu.sync_copy(idx_hbm.at[pl.ds(crow0, _CPT), :], idx_v)

    def group(g, carry):
        gathers = []
        for b in range(_GROUP):
            j = g * _GROUP + b
            gathers.append(
                pltpu.async_copy(table_hbm.at[idx_v.at[j]], rows_v.at[b],
                                 gsems.at[b])
            )
        copies = []
        for b in range(_GROUP):
            j = g * _GROUP + b
            gathers[b].wait()
            copies.append(
                pltpu.async_copy(
                    rows_v.at[b],
                    out_hbm.at[pl.ds((crow0 + j) * _CHUNK, _CHUNK), :],
                    csems.at[b])
            )
        for b in range(_GROUP):
            copies[b].wait()
        return carry

    lax.fori_loop(0, _NGROUP, group, 0)


@jax.jit
def _embedding_gather(idx2d, table):
    mesh = plsc.VectorSubcoreMesh(core_axis_name="c", subcore_axis_name="s")
    run = pl.kernel(
        _body,
        out_type=jax.ShapeDtypeStruct((_TOTAL, EMBEDDING_DIM), jnp.float32),
        mesh=mesh,
        scratch_types=[
            pltpu.VMEM((_CPT, _CHUNK), jnp.int32),
            pltpu.VMEM((_GROUP, _CHUNK, EMBEDDING_DIM), jnp.float32),
            pltpu.SemaphoreType.DMA((_GROUP,)),
            pltpu.SemaphoreType.DMA((_GROUP,)),
        ],
        compiler_params=pltpu.CompilerParams(use_tc_tiling_on_sc=False),
    )
    return run(table, idx2d)


def kernel(inputs, shared_weights):
    idx2d = inputs.astype(jnp.int32).reshape(_TOTAL // _CHUNK, _CHUNK)
    out = _embedding_gather(idx2d, shared_weights)
    return out.reshape(BATCH, SEQ_LEN, EMBEDDING_DIM)


# speedup vs baseline: 1.1116x; 1.0016x over previous
import functools

import jax
import jax.numpy as jnp
from jax import lax
from jax.experimental import pallas as pl
from jax.experimental.pallas import tpu as pltpu
from jax.experimental.pallas import tpu_sc as plsc

VOCAB_SIZE = 1000000
EMBEDDING_DIM = 32
BATCH = 16384
SEQ_LEN = 50

_NC = 2
_NS = 16
_NW = _NC * _NS

_TOTAL = BATCH * SEQ_LEN
_CHUNK = 256
_CPT = _TOTAL // (_NW * _CHUNK)
_NBUF = 10
_NGEN = _CPT // _NBUF


def _body(table_hbm, idx_hbm, out_hbm, idx_v, rows_v, gsems, csems):
    wid = lax.axis_index("s") * _NC + lax.axis_index("c")
    crow0 = wid * _CPT

    def gather(b, j):
        return pltpu.make_async_copy(table_hbm.at[idx_v.at[j]], rows_v.at[b],
                                     gsems.at[b])

    def putout(b, j):
        return pltpu.make_async_copy(
            rows_v.at[b], out_hbm.at[pl.ds((crow0 + j) * _CHUNK, _CHUNK), :],
            csems.at[b])

    pltpu.sync_copy(idx_hbm.at[pl.ds(crow0, _CPT), :], idx_v)

    for b in range(_NBUF):
        gather(b, b).start()

    def gen(g, carry):
        for b in range(_NBUF):
            j = g * _NBUF + b
            gather(b, j).wait()
            putout(b, j).start()
        for b in range(_NBUF):
            j = g * _NBUF + b
            putout(b, j).wait()
            gather(b, j + _NBUF).start()
        return carry

    lax.fori_loop(0, _NGEN - 1, gen, 0)

    g = _NGEN - 1
    for b in range(_NBUF):
        j = g * _NBUF + b
        gather(b, j).wait()
        putout(b, j).start()
    for b in range(_NBUF):
        putout(b, g * _NBUF + b).wait()


@jax.jit
def _embedding_gather(idx2d, table):
    mesh = plsc.VectorSubcoreMesh(core_axis_name="c", subcore_axis_name="s")
    run = pl.kernel(
        _body,
        out_type=jax.ShapeDtypeStruct((_TOTAL, EMBEDDING_DIM), jnp.float32),
        mesh=mesh,
        scratch_types=[
            pltpu.VMEM((_CPT, _CHUNK), jnp.int32),
            pltpu.VMEM((_NBUF, _CHUNK, EMBEDDING_DIM), jnp.float32),
            pltpu.SemaphoreType.DMA((_NBUF,)),
            pltpu.SemaphoreType.DMA((_NBUF,)),
        ],
        compiler_params=pltpu.CompilerParams(use_tc_tiling_on_sc=False),
    )
    return run(table, idx2d)


def kernel(inputs, shared_weights):
    idx2d = inputs.astype(jnp.int32).reshape(_TOTAL // _CHUNK, _CHUNK)
    out = _embedding_gather(idx2d, shared_weights)
    return out.reshape(BATCH, SEQ_LEN, EMBEDDING_DIM)


# speedup vs baseline: 1.4989x; 1.3484x over previous
import functools

import jax
import jax.numpy as jnp
from jax import lax
from jax.experimental import pallas as pl
from jax.experimental.pallas import tpu as pltpu
from jax.experimental.pallas import tpu_sc as plsc

VOCAB_SIZE = 1000000
EMBEDDING_DIM = 32
BATCH = 16384
SEQ_LEN = 50

_NC = 2
_NS = 16
_NW = _NC * _NS

_BPT = BATCH // _NW
_NBUF = 4
_NT = 2
_LANES = 16


def _body(table_hbm, idxT_hbm, out_hbm, idx_v, rows_v, tbuf, gsems, osems):
    wid = lax.axis_index("s") * _NC + lax.axis_index("c")
    b0 = wid * _BPT

    def gather(slot, s):
        return pltpu.make_async_copy(table_hbm.at[idx_v.at[s]],
                                     rows_v.at[slot], gsems.at[slot])

    def outcopy(ts, s):
        return pltpu.make_async_copy(tbuf.at[ts],
                                     out_hbm.at[s, :, pl.ds(b0, _BPT)],
                                     osems.at[ts])

    def transpose(slot, ts):
        rows = rows_v.at[slot]

        def percol(c, carry):
            cvec = jnp.full((_LANES,), 0, jnp.int32) + c
            for v in range(_BPT // _LANES):
                bvec = v * _LANES + lax.iota(jnp.int32, _LANES)
                vals = plsc.load_gather(rows, [bvec, cvec])
                tbuf[ts, c, pl.ds(v * _LANES, _LANES)] = vals
            return carry

        lax.fori_loop(0, EMBEDDING_DIM, percol, 0)

    pltpu.sync_copy(idxT_hbm.at[:, pl.ds(b0, _BPT)], idx_v)

    for s in range(_NBUF):
        gather(s, s).start()

    def group(g, carry):
        for k in range(_NBUF):
            s = g * _NBUF + k
            ts = k % _NT

            @pl.when(s < SEQ_LEN)
            def _():
                gather(k, s).wait()

                @pl.when(s >= _NT)
                def _():
                    outcopy(ts, s - _NT).wait()

                transpose(k, ts)
                outcopy(ts, s).start()

                @pl.when(s + _NBUF < SEQ_LEN)
                def _():
                    gather(k, s + _NBUF).start()

        return carry

    lax.fori_loop(0, (SEQ_LEN + _NBUF - 1) // _NBUF, group, 0)

    for s in (SEQ_LEN - 2, SEQ_LEN - 1):
        outcopy(s % _NT, s).wait()


@jax.jit
def _embedding_gather(idxT, table):
    mesh = plsc.VectorSubcoreMesh(core_axis_name="c", subcore_axis_name="s")
    run = pl.kernel(
        _body,
        out_type=jax.ShapeDtypeStruct((SEQ_LEN, EMBEDDING_DIM, BATCH),
                                      jnp.float32),
        mesh=mesh,
        scratch_types=[
            pltpu.VMEM((SEQ_LEN, _BPT), jnp.int32),
            pltpu.VMEM((_NBUF, _BPT, EMBEDDING_DIM), jnp.float32),
            pltpu.VMEM((_NT, EMBEDDING_DIM, _BPT), jnp.float32),
            pltpu.SemaphoreType.DMA((_NBUF,)),
            pltpu.SemaphoreType.DMA((_NT,)),
        ],
        compiler_params=pltpu.CompilerParams(use_tc_tiling_on_sc=False,
                                             needs_layout_passes=False),
    )
    return run(table, idxT)


def kernel(inputs, shared_weights):
    idxT = inputs.astype(jnp.int32).T
    out3 = _embedding_gather(idxT, shared_weights)
    return out3.transpose(2, 0, 1)


# speedup vs baseline: 1.9435x; 1.2966x over previous
import functools

import jax
import jax.numpy as jnp
from jax import lax
from jax.experimental import pallas as pl
from jax.experimental.pallas import tpu as pltpu
from jax.experimental.pallas import tpu_sc as plsc

VOCAB_SIZE = 1000000
EMBEDDING_DIM = 32
BATCH = 16384
SEQ_LEN = 50

_NC = 2
_NS = 16
_NW = _NC * _NS

_BPT = BATCH // _NW
_NBUF = 4
_NT = 2
_LANES = 16


def _body(table_hbm, idxT_hbm, out_hbm, idx_v, rows_v, gsems, osems):
    wid = lax.axis_index("s") * _NC + lax.axis_index("c")
    b0 = wid * _BPT

    def gather(slot, s):
        return pltpu.make_async_copy(table_hbm.at[idx_v.at[s]],
                                     rows_v.at[slot], gsems.at[slot])

    def outcopy(slot, s):
        return pltpu.make_async_copy(rows_v.at[slot],
                                     out_hbm.at[s, pl.ds(b0, _BPT), :],
                                     osems.at[slot])

    pltpu.sync_copy(idxT_hbm.at[:, pl.ds(b0, _BPT)], idx_v)

    for s in range(_NBUF):
        gather(s, s).start()

    def group(g, carry):
        for k in range(_NBUF):
            s = g * _NBUF + k

            @pl.when(s < SEQ_LEN)
            def _():
                gather(k, s).wait()
                outcopy(k, s).start()

                @pl.when(s + _NBUF < SEQ_LEN)
                def _():
                    outcopy(k, s).wait()
                    gather(k, s + _NBUF).start()

        return carry

    lax.fori_loop(0, (SEQ_LEN + _NBUF - 1) // _NBUF, group, 0)

    for k in range(_NBUF):
        outcopy(k, SEQ_LEN - _NBUF + k).wait()


@jax.jit
def _embedding_gather(idxT, table):
    mesh = plsc.VectorSubcoreMesh(core_axis_name="c", subcore_axis_name="s")
    run = pl.kernel(
        _body,
        out_type=jax.ShapeDtypeStruct((SEQ_LEN, BATCH, EMBEDDING_DIM),
                                      jnp.float32),
        mesh=mesh,
        scratch_types=[
            pltpu.VMEM((SEQ_LEN, _BPT), jnp.int32),
            pltpu.VMEM((_NBUF, _BPT, EMBEDDING_DIM), jnp.float32),
            pltpu.SemaphoreType.DMA((_NBUF,)),
            pltpu.SemaphoreType.DMA((_NBUF,)),
        ],
        compiler_params=pltpu.CompilerParams(use_tc_tiling_on_sc=False,
                                             needs_layout_passes=False),
    )
    return run(table, idxT)


def kernel(inputs, shared_weights):
    idxT = inputs.astype(jnp.int32).T
    out3 = _embedding_gather(idxT, shared_weights)
    return out3.transpose(1, 0, 2)


# speedup vs baseline: 2.6816x; 1.3798x over previous
import functools

import jax
import jax.numpy as jnp
from jax import lax
from jax.experimental import pallas as pl
from jax.experimental.pallas import tpu as pltpu
from jax.experimental.pallas import tpu_sc as plsc

VOCAB_SIZE = 1000000
EMBEDDING_DIM = 32
BATCH = 16384
SEQ_LEN = 50

_NC = 2
_NS = 16
_NW = _NC * _NS

_BPT = BATCH // _NW
_NBUF = 4
_NT = 2
_LANES = 16


def _body(table_hbm, idxT_hbm, out_hbm, idx_v, rows_v, tbuf, gsems, osems):
    wid = lax.axis_index("s") * _NC + lax.axis_index("c")
    b0 = wid * _BPT

    def gather(slot, s):
        return pltpu.make_async_copy(table_hbm.at[idx_v.at[s]],
                                     rows_v.at[slot], gsems.at[slot])

    def outcopy(ts, s):
        return pltpu.make_async_copy(tbuf.at[ts],
                                     out_hbm.at[s, :, pl.ds(b0, _BPT)],
                                     osems.at[ts])

    def transpose(slot, ts):
        rows = rows_v.at[slot]

        @functools.partial(plsc.parallel_loop, 0, EMBEDDING_DIM, unroll=2)
        def _(c):
            cvec = jnp.full((_LANES,), 0, jnp.int32) + c
            for v in range(_BPT // _LANES):
                bvec = v * _LANES + lax.iota(jnp.int32, _LANES)
                vals = plsc.load_gather(rows, [bvec, cvec])
                tbuf[ts, c, pl.ds(v * _LANES, _LANES)] = vals

    pltpu.sync_copy(idxT_hbm.at[:, pl.ds(b0, _BPT)], idx_v)

    for s in range(_NBUF):
        gather(s, s).start()

    def group(g, carry):
        for k in range(_NBUF):
            s = g * _NBUF + k
            ts = k % _NT

            @pl.when(s < SEQ_LEN)
            def _():
                gather(k, s).wait()

                @pl.when(s >= _NT)
                def _():
                    outcopy(ts, s - _NT).wait()

                transpose(k, ts)
                outcopy(ts, s).start()

                @pl.when(s + _NBUF < SEQ_LEN)
                def _():
                    gather(k, s + _NBUF).start()

        return carry

    lax.fori_loop(0, (SEQ_LEN + _NBUF - 1) // _NBUF, group, 0)

    for s in (SEQ_LEN - 2, SEQ_LEN - 1):
        outcopy(s % _NT, s).wait()


@jax.jit
def _embedding_gather(idxT, table):
    mesh = plsc.VectorSubcoreMesh(core_axis_name="c", subcore_axis_name="s")
    run = pl.kernel(
        _body,
        out_type=jax.ShapeDtypeStruct((SEQ_LEN, EMBEDDING_DIM, BATCH),
                                      jnp.float32),
        mesh=mesh,
        scratch_types=[
            pltpu.VMEM((SEQ_LEN, _BPT), jnp.int32),
            pltpu.VMEM((_NBUF, _BPT, EMBEDDING_DIM), jnp.float32),
            pltpu.VMEM((_NT, EMBEDDING_DIM, _BPT), jnp.float32),
            pltpu.SemaphoreType.DMA((_NBUF,)),
            pltpu.SemaphoreType.DMA((_NT,)),
        ],
        compiler_params=pltpu.CompilerParams(use_tc_tiling_on_sc=False,
                                             needs_layout_passes=False),
    )
    return run(table, idxT)


def kernel(inputs, shared_weights):
    idxT = inputs.astype(jnp.int32).T
    out3 = _embedding_gather(idxT, shared_weights)
    return out3.transpose(2, 0, 1)
